# baseline (speedup 1.0000x reference)
import functools
import math

import jax
import jax.numpy as jnp
from jax import lax
from jax.experimental import pallas as pl
from jax.experimental.pallas import tpu as pltpu
from jax.experimental.pallas import tpu_sc as plsc


_CHUNK = 80


@functools.lru_cache(maxsize=None)
def _make_gather2(n_rows, d, n_idx, dtype=jnp.float32):
  info = plsc.get_sparse_core_info()
  nc, ns = info.num_cores, info.num_subcores
  nw = nc * ns
  per_w = n_idx // nw
  assert n_idx % nw == 0 and per_w % _CHUNK == 0
  n_iter = per_w // _CHUNK
  mesh = plsc.VectorSubcoreMesh(core_axis_name="c", subcore_axis_name="s")

  @functools.partial(
      pl.kernel,
      out_type=(jax.ShapeDtypeStruct((n_idx, d), dtype),
                jax.ShapeDtypeStruct((n_idx, d), dtype)),
      mesh=mesh,
      compiler_params=pltpu.CompilerParams(use_tc_tiling_on_sc=False),
      scratch_types=[
          pltpu.VMEM((_CHUNK,), jnp.int32),
          pltpu.VMEM((_CHUNK,), jnp.int32),
          pltpu.VMEM((_CHUNK, d), dtype),
          pltpu.VMEM((_CHUNK, d), dtype),
          pltpu.SemaphoreType.DMA,
          pltpu.SemaphoreType.DMA,
          pltpu.SemaphoreType.DMA,
          pltpu.SemaphoreType.DMA,
      ],
  )
  def gather(table_hbm, idxa_hbm, idxb_hbm, outa_hbm, outb_hbm,
             ia_v, ib_v, ra_v, rb_v, sga, sgb, swa, swb):
    wid = lax.axis_index("s") * nc + lax.axis_index("c")
    base = wid * per_w

    def body(i, carry):
      off = base + i * _CHUNK
      pltpu.sync_copy(idxa_hbm.at[pl.ds(off, _CHUNK)], ia_v)
      pltpu.sync_copy(idxb_hbm.at[pl.ds(off, _CHUNK)], ib_v)
      ha = pltpu.async_copy(table_hbm.at[ia_v], ra_v, sga)
      hb = pltpu.async_copy(table_hbm.at[ib_v], rb_v, sgb)
      ha.wait()
      wa = pltpu.async_copy(ra_v, outa_hbm.at[pl.ds(off, _CHUNK)], swa)
      hb.wait()
      wb = pltpu.async_copy(rb_v, outb_hbm.at[pl.ds(off, _CHUNK)], swb)
      wa.wait()
      wb.wait()
      return carry

    lax.fori_loop(0, n_iter, body, 0)

  return gather


@functools.lru_cache(maxsize=None)
def _make_scatter_add(n_rows, d, dz, n_idx):
  info = plsc.get_sparse_core_info()
  nc, ns = info.num_cores, info.num_subcores
  per_w = n_idx // (nc * ns)
  assert n_idx % (nc * ns) == 0 and per_w % _CHUNK == 0
  n_iter = per_w // _CHUNK
  zchunks = n_rows // _CHUNK
  assert n_rows % _CHUNK == 0
  zouter = (zchunks + ns - 1) // ns
  mesh = plsc.VectorSubcoreMesh(core_axis_name="c", subcore_axis_name="s")

  @functools.partial(
      pl.kernel,
      out_type=(jax.ShapeDtypeStruct((nc, n_rows, d), jnp.float32),
                jax.ShapeDtypeStruct((nc, n_rows, dz), jnp.float32)),
      mesh=mesh,
      compiler_params=pltpu.CompilerParams(use_tc_tiling_on_sc=False),
      scratch_types=[
          pltpu.VMEM((_CHUNK,), jnp.int32),
          pltpu.VMEM((_CHUNK, d), jnp.float32),
          pltpu.VMEM((_CHUNK, dz), jnp.float32),
          pltpu.VMEM_SHARED((n_rows, d), jnp.float32),
          pltpu.VMEM_SHARED((n_rows, dz), jnp.float32),
          pltpu.SemaphoreType.DMA,
          pltpu.SemaphoreType.DMA,
          pltpu.SemaphoreType.DMA,
          pltpu.SemaphoreType.DMA,
          pltpu.SemaphoreType.DMA,
      ],
  )
  def scatter(x_hbm, e_hbm, idx_hbm, zeros_hbm, zerosz_hbm, out_hbm, outz_hbm,
              idx_v, rows_v, e_v, acc, accz, s1, s2, s3, s4, s5):
    cid = lax.axis_index("c")
    sid = lax.axis_index("s")
    wid = sid * nc + cid

    def zbody(i, carry):
      j = sid + i * ns

      @pl.when(j < zchunks)
      def _():
        pltpu.sync_copy(zeros_hbm.at[pl.ds(j * _CHUNK, _CHUNK)],
                        acc.at[pl.ds(j * _CHUNK, _CHUNK)])
        pltpu.sync_copy(zerosz_hbm.at[pl.ds(j * _CHUNK, _CHUNK)],
                        accz.at[pl.ds(j * _CHUNK, _CHUNK)])

      return carry

    lax.fori_loop(0, zouter, zbody, 0)
    plsc.subcore_barrier()

    base = wid * per_w

    def body(i, carry):
      off = base + i * _CHUNK
      h1 = pltpu.async_copy(idx_hbm.at[pl.ds(off, _CHUNK)], idx_v, s1)
      h2 = pltpu.async_copy(x_hbm.at[pl.ds(off, _CHUNK)], rows_v, s2)
      h3 = pltpu.async_copy(e_hbm.at[pl.ds(off, _CHUNK)], e_v, s3)
      h1.wait()
      h2.wait()
      h4 = pltpu.async_copy(rows_v, acc.at[idx_v], s4, add=True)
      h3.wait()
      h5 = pltpu.async_copy(e_v, accz.at[idx_v], s5, add=True)
      h4.wait()
      h5.wait()
      return carry

    lax.fori_loop(0, n_iter, body, 0)
    plsc.subcore_barrier()

    def obody(i, carry):
      j = sid + i * ns

      @pl.when(j < zchunks)
      def _():
        pltpu.sync_copy(acc.at[pl.ds(j * _CHUNK, _CHUNK)],
                        out_hbm.at[cid, pl.ds(j * _CHUNK, _CHUNK)])
        pltpu.sync_copy(accz.at[pl.ds(j * _CHUNK, _CHUNK)],
                        outz_hbm.at[cid, pl.ds(j * _CHUNK, _CHUNK)])

      return carry

    lax.fori_loop(0, zouter, obody, 0)

  return scatter


_ET = 1600
_NT = 1000


def _group_sum_mat(width, groups, group_width):
  r = lax.broadcasted_iota(jnp.int32, (width, groups), 0)
  c = lax.broadcasted_iota(jnp.int32, (width, groups), 1)
  return (r // group_width == c).astype(jnp.float32)


def _bcast_mat(groups, width, group_width):
  r = lax.broadcasted_iota(jnp.int32, (groups, width), 0)
  c = lax.broadcasted_iota(jnp.int32, (groups, width), 1)
  return (c // group_width == r).astype(jnp.float32)


def _bf(x):
  return x.astype(jnp.bfloat16)


def _edge_attn_body(scale, n_head, d_key, d_div,
                    ns, ef, nd, wt, wm, wb, ba, out, out_e):
  x = jnp.dot(_bf(ns[...]), _bf(wt[...]), preferred_element_type=jnp.float32)
  x += jnp.dot(_bf(ef[...]), _bf(wm[...]), preferred_element_type=jnp.float32)
  x += jnp.dot(_bf(nd[...]), _bf(wb[...]), preferred_element_type=jnp.float32)
  x += ba[...]
  nqk = n_head * d_key
  q = x[:, :nqk]
  k = x[:, nqk:2 * nqk]
  v = x[:, 2 * nqk:]
  prod = q * k
  gsum = _group_sum_mat(nqk, 16, d_key)
  dot = jnp.dot(prod, gsum, preferred_element_type=jnp.float32) * scale
  dot = jnp.clip(dot, -5.0, 5.0)
  head_mask = (lax.broadcasted_iota(jnp.int32, dot.shape, 1)
               < n_head).astype(jnp.float32)
  e16 = jnp.exp(dot) * head_mask
  bmat = _bcast_mat(16, n_head * d_div, d_div)
  evh = v * jnp.dot(e16, bmat, preferred_element_type=jnp.float32)
  out[...] = evh
  out_e[...] = e16


def _ln(x, g, b, eps=1e-6):
  m = jnp.mean(x, axis=-1, keepdims=True)
  c = x - m
  v = jnp.mean(c * c, axis=-1, keepdims=True)
  return c * lax.rsqrt(v + eps) * g + b


def _node_epi_body(n_head, d_div,
                   a0, a1, z0, z1, nf, wvl, bvl, ga, bba, w1, b1, gf, bf, w2,
                   b2, out, out16):
  u = a0[...] + a1[...]
  d_out = n_head * d_div
  z = z0[...] + z1[...]
  z = jnp.where(z == 0.0, 1.0, z)
  bmat = _bcast_mat(16, d_out, d_div)
  heads = u / jnp.dot(z, bmat, preferred_element_type=jnp.float32)
  v = jnp.dot(heads, wvl[...], preferred_element_type=jnp.float32) + bvl[...]
  v = nf[...] + v
  v = _ln(v, ga[...], bba[...])
  v2 = jnp.dot(v, w1[...], preferred_element_type=jnp.float32) + b1[...] + v
  v2 = _ln(v2, gf[...], bf[...])
  v2 = v2 * jax.nn.sigmoid(v2)
  nn = jnp.dot(v2, w2[...], preferred_element_type=jnp.float32) + b2[...]
  out[...] = nn
  out16[...] = nn.astype(jnp.bfloat16)


def _edge_epi_body(ns, ef, nd, wt, wm, wb, b3, ge, be, w4, b4, out):
  h = jnp.dot(_bf(ns[...]), _bf(wt[...]), preferred_element_type=jnp.float32)
  h += jnp.dot(_bf(ef[...]), _bf(wm[...]), preferred_element_type=jnp.float32)
  h += jnp.dot(_bf(nd[...]), _bf(wb[...]), preferred_element_type=jnp.float32)
  h += b3[...]
  h = _ln(h, ge[...], be[...])
  h = h * jax.nn.sigmoid(h)
  out[...] = jnp.dot(h, w4[...], preferred_element_type=jnp.float32) + b4[...]


def _full(shape):
  return pl.BlockSpec(shape, lambda i: (0,) * len(shape))


def _tiled(t, d):
  return pl.BlockSpec((t, d), lambda i: (i, 0))


def kernel(node_fea_in, edge_fea_in, edge_src, edge_dst, Wq, bq, Wk, bk, Wv,
           bv, W_vl, b_vl, g_a, b_a, W1, b1, g_ffn, b_ffn, W2, b2, W3, b3,
           g_e, b_e, W4, b4):
  n, d_node = node_fea_in.shape
  e, d_edge = edge_fea_in.shape
  n_head, d_cat, d_key = Wq.shape
  d_div = Wv.shape[-1]
  d_out = n_head * d_div
  d_out_edge = W4.shape[-1]
  scale = 1.0 / math.sqrt(d_cat)

  def _stack(w, lo, hi):
    return jnp.transpose(w[:, lo:hi, :], (1, 0, 2)).reshape(hi - lo, -1)

  def _part(lo, hi):
    return jnp.concatenate(
        [_stack(Wq, lo, hi), _stack(Wk, lo, hi), _stack(Wv, lo, hi)], axis=1)

  w_top = _part(0, d_node)
  w_mid = _part(d_node, d_node + d_edge)
  w_bot = _part(d_node + d_edge, d_cat)
  b_all = jnp.concatenate(
      [bq.reshape(-1), bk.reshape(-1), bv.reshape(-1)]).reshape(1, -1)
  d_proj = b_all.shape[-1]

  gather = _make_gather2(n, d_node, e, jnp.bfloat16)
  scatter = _make_scatter_add(n, d_out, 16, e)

  gs, gd = gather(node_fea_in.astype(jnp.bfloat16), edge_src, edge_dst)

  n_et = e // _ET
  contrib, e_arr = pl.pallas_call(
      functools.partial(_edge_attn_body, scale, n_head, d_key, d_div),
      grid=(n_et,),
      in_specs=[
          _tiled(_ET, d_node), _tiled(_ET, d_edge), _tiled(_ET, d_node),
          _full((d_node, d_proj)), _full((d_edge, d_proj)),
          _full((d_node, d_proj)), _full((1, d_proj)),
      ],
      out_specs=(_tiled(_ET, d_out), _tiled(_ET, 16)),
      out_shape=(jax.ShapeDtypeStruct((e, d_out), jnp.float32),
                 jax.ShapeDtypeStruct((e, 16), jnp.float32)),
  )(gs, edge_fea_in, gd, w_top, w_mid, w_bot, b_all)

  zeros = jnp.zeros((n, d_out), jnp.float32)
  zerosz = jnp.zeros((n, 16), jnp.float32)
  acc, accz = scatter(contrib, e_arr, edge_dst, zeros, zerosz)

  row = lambda x: x.reshape(1, -1)
  n_nt = n // _NT
  new_node = pl.pallas_call(
      functools.partial(_node_epi_body, n_head, d_div),
      grid=(n_nt,),
      in_specs=[
          _tiled(_NT, d_out), _tiled(_NT, d_out),
          _tiled(_NT, 16), _tiled(_NT, 16),
          _tiled(_NT, d_node),
          _full((d_out, d_node)), _full((1, d_node)),
          _full((1, d_node)), _full((1, d_node)),
          _full((d_node, d_node)), _full((1, d_node)),
          _full((1, d_node)), _full((1, d_node)),
          _full((d_node, d_node)), _full((1, d_node)),
      ],
      out_specs=(_tiled(_NT, d_node), _tiled(_NT, d_node)),
      out_shape=(jax.ShapeDtypeStruct((n, d_node), jnp.float32),
                 jax.ShapeDtypeStruct((n, d_node), jnp.bfloat16)),
  )(acc[0], acc[1], accz[0], accz[1], node_fea_in, W_vl, row(b_vl), row(g_a),
    row(b_a), W1, row(b1), row(g_ffn), row(b_ffn), W2, row(b2))
  new_node, new_node16 = new_node

  hs, hd = gather(new_node16, edge_src, edge_dst)

  ef_out = pl.pallas_call(
      _edge_epi_body,
      grid=(n_et,),
      in_specs=[
          _tiled(_ET, d_node), _tiled(_ET, d_edge), _tiled(_ET, d_node),
          _full((d_node, d_out_edge)), _full((d_edge, d_out_edge)),
          _full((d_node, d_out_edge)), _full((1, d_out_edge)),
          _full((1, d_out_edge)), _full((1, d_out_edge)),
          _full((d_out_edge, d_out_edge)), _full((1, d_out_edge)),
      ],
      out_specs=_tiled(_ET, d_out_edge),
      out_shape=jax.ShapeDtypeStruct((e, d_out_edge), jnp.float32),
  )(hs, edge_fea_in, hd, W3[:d_node], W3[d_node:d_node + d_edge],
    W3[d_node + d_edge:], row(b3), row(g_e), row(b_e), W4, row(b4))

  return new_node, ef_out


# speedup vs baseline: 1.5054x; 1.5054x over previous
import functools
import math

import jax
import jax.numpy as jnp
from jax import lax
from jax.experimental import pallas as pl
from jax.experimental.pallas import tpu as pltpu
from jax.experimental.pallas import tpu_sc as plsc


_CHUNK = 80


@functools.lru_cache(maxsize=None)
def _make_gather2(n_rows, d, n_idx, dtype=jnp.float32):
  info = plsc.get_sparse_core_info()
  nc, ns = info.num_cores, info.num_subcores
  nw = nc * ns
  per_w = n_idx // nw
  assert n_idx % nw == 0 and per_w % _CHUNK == 0
  n_iter = per_w // _CHUNK
  mesh = plsc.VectorSubcoreMesh(core_axis_name="c", subcore_axis_name="s")

  @functools.partial(
      pl.kernel,
      out_type=(jax.ShapeDtypeStruct((n_idx, d), dtype),
                jax.ShapeDtypeStruct((n_idx, d), dtype)),
      mesh=mesh,
      scratch_types=[
          pltpu.VMEM((_CHUNK,), jnp.int32),
          pltpu.VMEM((_CHUNK,), jnp.int32),
          pltpu.VMEM((_CHUNK, d), dtype),
          pltpu.VMEM((_CHUNK, d), dtype),
          pltpu.SemaphoreType.DMA,
          pltpu.SemaphoreType.DMA,
          pltpu.SemaphoreType.DMA,
          pltpu.SemaphoreType.DMA,
      ],
  )
  def gather(table_hbm, idxa_hbm, idxb_hbm, outa_hbm, outb_hbm,
             ia_v, ib_v, ra_v, rb_v, sga, sgb, swa, swb):
    wid = lax.axis_index("s") * nc + lax.axis_index("c")
    base = wid * per_w

    def body(i, carry):
      off = base + i * _CHUNK
      pltpu.sync_copy(idxa_hbm.at[pl.ds(off, _CHUNK)], ia_v)
      pltpu.sync_copy(idxb_hbm.at[pl.ds(off, _CHUNK)], ib_v)
      ha = pltpu.async_copy(table_hbm.at[ia_v], ra_v, sga)
      hb = pltpu.async_copy(table_hbm.at[ib_v], rb_v, sgb)
      ha.wait()
      wa = pltpu.async_copy(ra_v, outa_hbm.at[pl.ds(off, _CHUNK)], swa)
      hb.wait()
      wb = pltpu.async_copy(rb_v, outb_hbm.at[pl.ds(off, _CHUNK)], swb)
      wa.wait()
      wb.wait()
      return carry

    lax.fori_loop(0, n_iter, body, 0)

  return gather


@functools.lru_cache(maxsize=None)
def _make_scatter_add(n_rows, d, dz, n_idx):
  info = plsc.get_sparse_core_info()
  nc, ns = info.num_cores, info.num_subcores
  per_w = n_idx // (nc * ns)
  assert n_idx % (nc * ns) == 0 and per_w % _CHUNK == 0
  n_iter = per_w // _CHUNK
  zchunks = n_rows // _CHUNK
  assert n_rows % _CHUNK == 0
  zouter = (zchunks + ns - 1) // ns
  mesh = plsc.VectorSubcoreMesh(core_axis_name="c", subcore_axis_name="s")

  @functools.partial(
      pl.kernel,
      out_type=(jax.ShapeDtypeStruct((nc, n_rows, d), jnp.float32),
                jax.ShapeDtypeStruct((nc, n_rows, dz), jnp.float32)),
      mesh=mesh,
      compiler_params=pltpu.CompilerParams(use_tc_tiling_on_sc=False),
      scratch_types=[
          pltpu.VMEM((_CHUNK,), jnp.int32),
          pltpu.VMEM((_CHUNK, d), jnp.float32),
          pltpu.VMEM((_CHUNK, dz), jnp.float32),
          pltpu.VMEM_SHARED((n_rows, d), jnp.float32),
          pltpu.VMEM_SHARED((n_rows, dz), jnp.float32),
          pltpu.SemaphoreType.DMA,
          pltpu.SemaphoreType.DMA,
          pltpu.SemaphoreType.DMA,
          pltpu.SemaphoreType.DMA,
          pltpu.SemaphoreType.DMA,
      ],
  )
  def scatter(x_hbm, e_hbm, idx_hbm, zeros_hbm, zerosz_hbm, out_hbm, outz_hbm,
              idx_v, rows_v, e_v, acc, accz, s1, s2, s3, s4, s5):
    cid = lax.axis_index("c")
    sid = lax.axis_index("s")
    wid = sid * nc + cid

    def zbody(i, carry):
      j = sid + i * ns

      @pl.when(j < zchunks)
      def _():
        pltpu.sync_copy(zeros_hbm.at[pl.ds(j * _CHUNK, _CHUNK)],
                        acc.at[pl.ds(j * _CHUNK, _CHUNK)])
        pltpu.sync_copy(zerosz_hbm.at[pl.ds(j * _CHUNK, _CHUNK)],
                        accz.at[pl.ds(j * _CHUNK, _CHUNK)])

      return carry

    lax.fori_loop(0, zouter, zbody, 0)
    plsc.subcore_barrier()

    base = wid * per_w

    def body(i, carry):
      off = base + i * _CHUNK
      h1 = pltpu.async_copy(idx_hbm.at[pl.ds(off, _CHUNK)], idx_v, s1)
      h2 = pltpu.async_copy(x_hbm.at[pl.ds(off, _CHUNK)], rows_v, s2)
      h3 = pltpu.async_copy(e_hbm.at[pl.ds(off, _CHUNK)], e_v, s3)
      h1.wait()
      h2.wait()
      h4 = pltpu.async_copy(rows_v, acc.at[idx_v], s4, add=True)
      h3.wait()
      h5 = pltpu.async_copy(e_v, accz.at[idx_v], s5, add=True)
      h4.wait()
      h5.wait()
      return carry

    lax.fori_loop(0, n_iter, body, 0)
    plsc.subcore_barrier()

    def obody(i, carry):
      j = sid + i * ns

      @pl.when(j < zchunks)
      def _():
        pltpu.sync_copy(acc.at[pl.ds(j * _CHUNK, _CHUNK)],
                        out_hbm.at[cid, pl.ds(j * _CHUNK, _CHUNK)])
        pltpu.sync_copy(accz.at[pl.ds(j * _CHUNK, _CHUNK)],
                        outz_hbm.at[cid, pl.ds(j * _CHUNK, _CHUNK)])

      return carry

    lax.fori_loop(0, zouter, obody, 0)

  return scatter


_ET = 1600
_NT = 1000


def _group_sum_mat(width, groups, group_width):
  r = lax.broadcasted_iota(jnp.int32, (width, groups), 0)
  c = lax.broadcasted_iota(jnp.int32, (width, groups), 1)
  return (r // group_width == c).astype(jnp.float32)


def _bcast_mat(groups, width, group_width):
  r = lax.broadcasted_iota(jnp.int32, (groups, width), 0)
  c = lax.broadcasted_iota(jnp.int32, (groups, width), 1)
  return (c // group_width == r).astype(jnp.float32)


def _bf(x):
  return x.astype(jnp.bfloat16)


def _edge_attn_body(scale, n_head, d_key, d_div,
                    ns, ef, nd, wt, wm, wb, ba, out, out_e):
  x = jnp.dot(_bf(ns[...]), _bf(wt[...]), preferred_element_type=jnp.float32)
  x += jnp.dot(_bf(ef[...]), _bf(wm[...]), preferred_element_type=jnp.float32)
  x += jnp.dot(_bf(nd[...]), _bf(wb[...]), preferred_element_type=jnp.float32)
  x += ba[...]
  nqk = n_head * d_key
  q = x[:, :nqk]
  k = x[:, nqk:2 * nqk]
  v = x[:, 2 * nqk:]
  prod = q * k
  gsum = _group_sum_mat(nqk, 16, d_key)
  dot = jnp.dot(prod, gsum, preferred_element_type=jnp.float32) * scale
  dot = jnp.clip(dot, -5.0, 5.0)
  head_mask = (lax.broadcasted_iota(jnp.int32, dot.shape, 1)
               < n_head).astype(jnp.float32)
  e16 = jnp.exp(dot) * head_mask
  bmat = _bcast_mat(16, n_head * d_div, d_div)
  evh = v * jnp.dot(e16, bmat, preferred_element_type=jnp.float32)
  out[...] = evh
  out_e[...] = e16


def _ln(x, g, b, eps=1e-6):
  m = jnp.mean(x, axis=-1, keepdims=True)
  c = x - m
  v = jnp.mean(c * c, axis=-1, keepdims=True)
  return c * lax.rsqrt(v + eps) * g + b


def _node_epi_body(n_head, d_div,
                   a0, a1, z0, z1, nf, wvl, bvl, ga, bba, w1, b1, gf, bf, w2,
                   b2, out):
  u = a0[...] + a1[...]
  d_out = n_head * d_div
  z = z0[...] + z1[...]
  z = jnp.where(z == 0.0, 1.0, z)
  bmat = _bcast_mat(16, d_out, d_div)
  heads = u / jnp.dot(z, bmat, preferred_element_type=jnp.float32)
  v = jnp.dot(heads, wvl[...], preferred_element_type=jnp.float32) + bvl[...]
  v = nf[...] + v
  v = _ln(v, ga[...], bba[...])
  v2 = jnp.dot(v, w1[...], preferred_element_type=jnp.float32) + b1[...] + v
  v2 = _ln(v2, gf[...], bf[...])
  v2 = v2 * jax.nn.sigmoid(v2)
  out[...] = jnp.dot(v2, w2[...], preferred_element_type=jnp.float32) + b2[...]


def _edge_epi_body(ns, ef, nd, wt, wm, wb, b3, ge, be, w4, b4, out):
  h = jnp.dot(_bf(ns[...]), _bf(wt[...]), preferred_element_type=jnp.float32)
  h += jnp.dot(_bf(ef[...]), _bf(wm[...]), preferred_element_type=jnp.float32)
  h += jnp.dot(_bf(nd[...]), _bf(wb[...]), preferred_element_type=jnp.float32)
  h += b3[...]
  h = _ln(h, ge[...], be[...])
  h = h * jax.nn.sigmoid(h)
  out[...] = jnp.dot(h, w4[...], preferred_element_type=jnp.float32) + b4[...]


def _full(shape):
  return pl.BlockSpec(shape, lambda i: (0,) * len(shape))


def _tiled(t, d):
  return pl.BlockSpec((t, d), lambda i: (i, 0))


def kernel(node_fea_in, edge_fea_in, edge_src, edge_dst, Wq, bq, Wk, bk, Wv,
           bv, W_vl, b_vl, g_a, b_a, W1, b1, g_ffn, b_ffn, W2, b2, W3, b3,
           g_e, b_e, W4, b4):
  n, d_node = node_fea_in.shape
  e, d_edge = edge_fea_in.shape
  n_head, d_cat, d_key = Wq.shape
  d_div = Wv.shape[-1]
  d_out = n_head * d_div
  d_out_edge = W4.shape[-1]
  scale = 1.0 / math.sqrt(d_cat)

  def _stack(w, lo, hi):
    return jnp.transpose(w[:, lo:hi, :], (1, 0, 2)).reshape(hi - lo, -1)

  def _part(lo, hi):
    return jnp.concatenate(
        [_stack(Wq, lo, hi), _stack(Wk, lo, hi), _stack(Wv, lo, hi)], axis=1)

  w_top = _part(0, d_node)
  w_mid = _part(d_node, d_node + d_edge)
  w_bot = _part(d_node + d_edge, d_cat)
  b_all = jnp.concatenate(
      [bq.reshape(-1), bk.reshape(-1), bv.reshape(-1)]).reshape(1, -1)
  d_proj = b_all.shape[-1]

  gather = _make_gather2(n, d_node, e)
  scatter = _make_scatter_add(n, d_out, 16, e)

  gs, gd = gather(node_fea_in, edge_src, edge_dst)

  n_et = e // _ET
  contrib, e_arr = pl.pallas_call(
      functools.partial(_edge_attn_body, scale, n_head, d_key, d_div),
      grid=(n_et,),
      in_specs=[
          _tiled(_ET, d_node), _tiled(_ET, d_edge), _tiled(_ET, d_node),
          _full((d_node, d_proj)), _full((d_edge, d_proj)),
          _full((d_node, d_proj)), _full((1, d_proj)),
      ],
      out_specs=(_tiled(_ET, d_out), _tiled(_ET, 16)),
      out_shape=(jax.ShapeDtypeStruct((e, d_out), jnp.float32),
                 jax.ShapeDtypeStruct((e, 16), jnp.float32)),
  )(gs, edge_fea_in, gd, w_top, w_mid, w_bot, b_all)

  zeros = jnp.zeros((n, d_out), jnp.float32)
  zerosz = jnp.zeros((n, 16), jnp.float32)
  acc, accz = scatter(contrib, e_arr, edge_dst, zeros, zerosz)

  row = lambda x: x.reshape(1, -1)
  n_nt = n // _NT
  new_node = pl.pallas_call(
      functools.partial(_node_epi_body, n_head, d_div),
      grid=(n_nt,),
      in_specs=[
          _tiled(_NT, d_out), _tiled(_NT, d_out),
          _tiled(_NT, 16), _tiled(_NT, 16),
          _tiled(_NT, d_node),
          _full((d_out, d_node)), _full((1, d_node)),
          _full((1, d_node)), _full((1, d_node)),
          _full((d_node, d_node)), _full((1, d_node)),
          _full((1, d_node)), _full((1, d_node)),
          _full((d_node, d_node)), _full((1, d_node)),
      ],
      out_specs=_tiled(_NT, d_node),
      out_shape=jax.ShapeDtypeStruct((n, d_node), jnp.float32),
  )(acc[0], acc[1], accz[0], accz[1], node_fea_in, W_vl, row(b_vl), row(g_a),
    row(b_a), W1, row(b1), row(g_ffn), row(b_ffn), W2, row(b2))

  hs, hd = gather(new_node, edge_src, edge_dst)

  ef_out = pl.pallas_call(
      _edge_epi_body,
      grid=(n_et,),
      in_specs=[
          _tiled(_ET, d_node), _tiled(_ET, d_edge), _tiled(_ET, d_node),
          _full((d_node, d_out_edge)), _full((d_edge, d_out_edge)),
          _full((d_node, d_out_edge)), _full((1, d_out_edge)),
          _full((1, d_out_edge)), _full((1, d_out_edge)),
          _full((d_out_edge, d_out_edge)), _full((1, d_out_edge)),
      ],
      out_specs=_tiled(_ET, d_out_edge),
      out_shape=jax.ShapeDtypeStruct((e, d_out_edge), jnp.float32),
  )(hs, edge_fea_in, hd, W3[:d_node], W3[d_node:d_node + d_edge],
    W3[d_node + d_edge:], row(b3), row(g_e), row(b_e), W4, row(b4))

  return new_node, ef_out
